# baseline (speedup 1.0000x reference)
import jax
import jax.numpy as jnp
from jax import lax
from jax.experimental import pallas as pl
from jax.experimental.pallas import tpu as pltpu
from jax.experimental.pallas import tpu_sc as plsc

_ROWS = 1_000_000
_DIM = 16
_NUM_CORES = 2
_NUM_WORKERS = 32
_LANES_PER_W = 31232
_CHUNK = 7808
_NCHUNK = _LANES_PER_W // _CHUNK
_TAIL_OFF = _LANES_PER_W * _NUM_WORKERS
_DUS_OFF = _TAIL_OFF + 4 * 128
_DUS = _ROWS - _DUS_OFF


def _copy_body(u_hbm, i_hbm, ou, oi, b0, b1, si0, si1, so0, so1):
    wid = lax.axis_index("s") * _NUM_CORES + lax.axis_index("c")
    base = wid * _LANES_PER_W
    bufs = (b0, b1)
    sin = (si0, si1)
    sout = (so0, so1)
    pending = [None, None]
    k = 0
    for tbl_in, tbl_out in ((u_hbm, ou), (i_hbm, oi)):
        for h in (0, 8):
            for c in range(_NCHUNK):
                slot = k % 2
                if pending[slot] is not None:
                    pending[slot].wait()
                sl = (pl.ds(h, 8), pl.ds(base + c * _CHUNK, _CHUNK))
                cin = pltpu.make_async_copy(tbl_in.at[sl], bufs[slot], sin[slot])
                cin.start()
                cin.wait()
                cout = pltpu.make_async_copy(bufs[slot], tbl_out.at[sl], sout[slot])
                cout.start()
                pending[slot] = cout
                k += 1
    for p in pending:
        if p is not None:
            p.wait()

    @pl.when(wid < 4)
    def _extra():
        off = _TAIL_OFF + wid * 128
        for tbl_in, tbl_out in ((u_hbm, ou), (i_hbm, oi)):
            for h in (0, 8):
                sl = (pl.ds(h, 8), pl.ds(off, 128))
                tb = b0.at[:, pl.ds(0, 128)]
                cin = pltpu.make_async_copy(tbl_in.at[sl], tb, si0)
                cin.start()
                cin.wait()
                cout = pltpu.make_async_copy(tb, tbl_out.at[sl], so0)
                cout.start()
                cout.wait()


def kernel(user_table, item_table):
    f = pl.kernel(
        _copy_body,
        out_type=(
            jax.ShapeDtypeStruct((_DIM, _ROWS), user_table.dtype),
            jax.ShapeDtypeStruct((_DIM, _ROWS), item_table.dtype),
        ),
        mesh=plsc.VectorSubcoreMesh(core_axis_name="c", subcore_axis_name="s"),
        scratch_types=[
            pltpu.VMEM((8, _CHUNK), jnp.float32),
            pltpu.VMEM((8, _CHUNK), jnp.float32),
            pltpu.SemaphoreType.DMA,
            pltpu.SemaphoreType.DMA,
            pltpu.SemaphoreType.DMA,
            pltpu.SemaphoreType.DMA,
        ],
    )
    ut, it = user_table.T, item_table.T
    out = f(ut, it)
    ou = lax.dynamic_update_slice(out[0], ut[:, _DUS_OFF:], (0, _DUS_OFF))
    oi = lax.dynamic_update_slice(out[1], it[:, _DUS_OFF:], (0, _DUS_OFF))
    return (ou.T, oi.T)


# speedup vs baseline: 1.4392x; 1.4392x over previous
import jax
import jax.numpy as jnp
from jax import lax
from jax.experimental import pallas as pl
from jax.experimental.pallas import tpu as pltpu

_ROWS = 1_000_000
_DIM = 16
_BLK = 118784
_GRID = (_ROWS + _BLK - 1) // _BLK


def _copy_body(u_ref, i_ref, ou_ref, oi_ref):
    ou_ref[...] = u_ref[...]
    oi_ref[...] = i_ref[...]


def kernel(user_table, item_table):
    spec = pl.BlockSpec((_DIM, _BLK), lambda k: (0, k))
    out = pl.pallas_call(
        _copy_body,
        grid=(_GRID,),
        in_specs=[spec, spec],
        out_specs=[spec, spec],
        out_shape=[
            jax.ShapeDtypeStruct((_DIM, _ROWS), user_table.dtype),
            jax.ShapeDtypeStruct((_DIM, _ROWS), item_table.dtype),
        ],
    )(user_table.T, item_table.T)
    return (out[0].T, out[1].T)
